# baseline (speedup 1.0000x reference)
import functools

import jax
import jax.numpy as jnp
from jax import lax
from jax.experimental import pallas as pl
from jax.experimental.pallas import tpu as pltpu
from jax.experimental.pallas import tpu_sc as plsc

_CHUNK = 16
_NBUF = 4


def _make_sc_gather(B, D):
    info = plsc.get_sparse_core_info()
    NC, NS = info.num_cores, info.num_subcores
    NW = NC * NS
    assert B % NW == 0 and D % 32 == 0
    b_per_w = B // NW
    C = _CHUNK
    NBUF = _NBUF
    assert b_per_w % (C * NBUF) == 0
    n_chunks = b_per_w // C
    H = D // 2

    mesh = plsc.VectorSubcoreMesh(core_axis_name="c", subcore_axis_name="s")

    @functools.partial(
        pl.kernel,
        mesh=mesh,
        out_type=jax.ShapeDtypeStruct((B, D), jnp.float32),
        scratch_types=[
            pltpu.VMEM((n_chunks, C), jnp.int32),
            pltpu.VMEM((NBUF, C, H), jnp.int32),
            pltpu.VMEM((NBUF, C, D), jnp.float32),
            pltpu.SemaphoreType.DMA((NBUF,)),
            pltpu.SemaphoreType.DMA((NBUF,)),
        ],
    )
    def gather_kernel(idx_hbm, table_hbm, out_hbm, idx_v, packed_v, rows_v,
                      gsem, wsem):
        wid = lax.axis_index("s") * NC + lax.axis_index("c")
        base = wid * b_per_w
        pltpu.sync_copy(idx_hbm.at[wid], idx_v)

        def gather_desc(c, b):
            return pltpu.make_async_copy(table_hbm.at[idx_v.at[c]],
                                         packed_v.at[b], gsem.at[b])

        def wb_desc(c, b):
            return pltpu.make_async_copy(rows_v.at[b],
                                         out_hbm.at[pl.ds(base + c * C, C)],
                                         wsem.at[b])

        hi_mask = jnp.int32(-65536)

        def widen_chunk(b):
            @plsc.parallel_loop(0, C, unroll=2)
            def _(r):
                for j in range(H // 16):
                    w = packed_v[b, r, pl.ds(j * 16, 16)]
                    lo = lax.bitcast_convert_type(lax.shift_left(w, 16),
                                                  jnp.float32)
                    hi = lax.bitcast_convert_type(
                        lax.bitwise_and(w, hi_mask), jnp.float32)
                    rows_v[b, r, pl.ds(j * 16, 16)] = lo
                    rows_v[b, r, pl.ds(H + j * 16, 16)] = hi

        for b in range(NBUF - 1):
            gather_desc(b, b).start()

        def body(g, carry):
            for b in range(NBUF):
                c = g * NBUF + b
                gather_desc(c, b).wait()
                nxt = c + NBUF - 1

                @pl.when(nxt < n_chunks)
                def _():
                    gather_desc(nxt, (b + NBUF - 1) % NBUF).start()

                @pl.when(c >= NBUF)
                def _():
                    wb_desc(c - NBUF, b).wait()

                widen_chunk(b)
                wb_desc(c, b).start()
            return carry

        lax.fori_loop(0, n_chunks // NBUF, body, 0)

        for j in range(NBUF):
            c = n_chunks - NBUF + j
            wb_desc(c, c % NBUF).wait()

    return gather_kernel


@jax.jit
def kernel(positions, weight):
    n_rows, d = weight.shape
    bsz, seq = positions.shape
    B = bsz * seq
    info = plsc.get_sparse_core_info()
    NW = info.num_cores * info.num_subcores
    C = _CHUNK
    idx = positions.reshape(NW, B // (NW * C), C).astype(jnp.int32)
    h = d // 2
    w_pairs = jnp.stack([weight[:, :h], weight[:, h:]], axis=-1)
    w_packed = lax.bitcast_convert_type(
        w_pairs.astype(jnp.bfloat16), jnp.int32)
    out = _make_sc_gather(B, d)(idx, w_packed)
    return out.reshape(bsz, seq, d)


# speedup vs baseline: 1.3180x; 1.3180x over previous
import functools

import jax
import jax.numpy as jnp
from jax import lax
from jax.experimental import pallas as pl
from jax.experimental.pallas import tpu as pltpu
from jax.experimental.pallas import tpu_sc as plsc


_CHUNK = 8
_NBUF = 8
_LOOK = 4


def _make_sc_gather(B, D, n_rows):
    info = plsc.get_sparse_core_info()
    NC, NS = info.num_cores, info.num_subcores
    NW = NC * NS
    assert B % NW == 0
    b_per_w = B // NW
    C = _CHUNK
    NBUF = _NBUF
    assert b_per_w % (C * NBUF) == 0
    n_chunks = b_per_w // C

    mesh = plsc.VectorSubcoreMesh(core_axis_name="c", subcore_axis_name="s")

    @functools.partial(
        pl.kernel,
        mesh=mesh,
        out_type=jax.ShapeDtypeStruct((B, D), jnp.float32),
        scratch_types=[
            pltpu.VMEM((n_chunks, C), jnp.int32),
            pltpu.VMEM((NBUF, C, D), jnp.float32),
            pltpu.SemaphoreType.DMA((NBUF,)),
            pltpu.SemaphoreType.DMA((NBUF,)),
        ],
    )
    def gather_kernel(idx_hbm, table_hbm, out_hbm, idx_v, rows_v, gsem, wsem):
        wid = lax.axis_index("s") * NC + lax.axis_index("c")
        base = wid * b_per_w
        pltpu.sync_copy(idx_hbm.at[wid], idx_v)

        def gather_desc(c, b):
            return pltpu.make_async_copy(table_hbm.at[idx_v.at[c]],
                                         rows_v.at[b], gsem.at[b])

        def wb_desc(c, b):
            return pltpu.make_async_copy(rows_v.at[b],
                                         out_hbm.at[pl.ds(base + c * C, C)],
                                         wsem.at[b])

        LOOK = _LOOK
        LAG = NBUF - LOOK

        for b in range(LOOK):
            gather_desc(b, b).start()

        def body(g, carry):
            for b in range(NBUF):
                c = g * NBUF + b
                gather_desc(c, b).wait()
                wb_desc(c, b).start()
                nxt = c + LOOK
                nb = (b + LOOK) % NBUF

                @pl.when(nxt < n_chunks)
                def _():
                    @pl.when(c >= LAG)
                    def _():
                        wb_desc(c - LAG, nb).wait()

                    gather_desc(nxt, nb).start()
            return carry

        lax.fori_loop(0, n_chunks // NBUF, body, 0)

        for j in range(NBUF):
            c = n_chunks - NBUF + j
            wb_desc(c, c % NBUF).wait()

    return gather_kernel


@jax.jit
def kernel(positions, weight):
    n_rows, d = weight.shape
    bsz, seq = positions.shape
    B = bsz * seq
    info = plsc.get_sparse_core_info()
    NW = info.num_cores * info.num_subcores
    C = _CHUNK
    idx = positions.reshape(NW, B // (NW * C), C).astype(jnp.int32)
    out = _make_sc_gather(B, d, n_rows)(idx, weight)
    return out.reshape(bsz, seq, d)
